# Initial kernel scaffold; baseline (speedup 1.0000x reference)
import jax
import jax.numpy as jnp
from jax.experimental import pallas as pl


def kernel(x, x_0, edge_index, edge_weight, W1):
    raise NotImplementedError("write your pallas kernel here")


# speedup vs baseline: 4.0818x; 4.0818x over previous
import functools
import math

import jax
import jax.numpy as jnp
from jax import lax
from jax.experimental import pallas as pl
from jax.experimental.pallas import tpu as pltpu
from jax.experimental.pallas import tpu_sc as plsc

_N = 10000
_D = 128
_E = 320000
_ALPHA = 0.1
_BETA = float(math.log(0.5 / 2.0 + 1.0))

_NC = 2
_NS = 16
_NW = _NC * _NS
_L = 16

_EPW = _E // _NW
_W = 80
_NWIN = _EPW // _W
_RPT = 624
_TAIL = _N - _NS * _RPT


def _splat(vec, i):
    idx = jnp.full((_L,), i, jnp.int32)
    dnums = lax.GatherDimensionNumbers(
        offset_dims=(), collapsed_slice_dims=(0,), start_index_map=(0,))
    return lax.gather(vec, idx[:, None], dnums, (1,),
                      mode=lax.GatherScatterMode.PROMISE_IN_BOUNDS)


def _sc_gather_scatter(x, src, dst, ew, zeros):
    mesh = plsc.VectorSubcoreMesh(
        core_axis_name="c", subcore_axis_name="s",
        num_cores=_NC, num_subcores=_NS)

    @functools.partial(
        pl.kernel,
        out_type=jax.ShapeDtypeStruct((_NC, _N, _D), jnp.float32),
        mesh=mesh,
        scratch_types=[
            pltpu.VMEM((_W,), jnp.int32),
            pltpu.VMEM((_W,), jnp.int32),
            pltpu.VMEM((_W,), jnp.float32),
            pltpu.VMEM((_W, _D), jnp.float32),
            pltpu.VMEM_SHARED((_N, _D), jnp.float32),
            pltpu.SemaphoreType.DMA,
        ],
    )
    def k(x_hbm, src_hbm, dst_hbm, ew_hbm, z_hbm, out_hbm,
          src_v, dst_v, w_v, rows_v, agg_sh, sem):
        c = lax.axis_index("c")
        s = lax.axis_index("s")
        gwid = s * _NC + c

        pltpu.sync_copy(z_hbm.at[pl.ds(s * _RPT, _RPT)],
                        agg_sh.at[pl.ds(s * _RPT, _RPT)])

        @pl.when(s == _NS - 1)
        def _zero_tail():
            pltpu.sync_copy(z_hbm.at[pl.ds(_NS * _RPT, _TAIL)],
                            agg_sh.at[pl.ds(_NS * _RPT, _TAIL)])

        plsc.subcore_barrier()

        def win_body(kk, carry):
            base = gwid * _EPW + kk * _W
            pltpu.sync_copy(src_hbm.at[pl.ds(base, _W)], src_v)
            pltpu.sync_copy(dst_hbm.at[pl.ds(base, _W)], dst_v)
            pltpu.sync_copy(ew_hbm.at[pl.ds(base, _W)], w_v)
            pltpu.async_copy(x_hbm.at[src_v], rows_v, sem).wait()

            def grp_body(i, carry2):
                w16 = w_v[pl.ds(i * _L, _L)]
                for e in range(_L):
                    ws = _splat(w16, e)
                    row = i * _L + e
                    for j in range(_D // _L):
                        rows_v[row, pl.ds(j * _L, _L)] = (
                            rows_v[row, pl.ds(j * _L, _L)] * ws)
                return carry2

            lax.fori_loop(0, _W // _L, grp_body, 0)
            pltpu.sync_copy(rows_v, agg_sh.at[dst_v], add=True)
            return carry

        lax.fori_loop(0, _NWIN, win_body, 0)

        plsc.subcore_barrier()
        pltpu.sync_copy(agg_sh.at[pl.ds(s * _RPT, _RPT)],
                        out_hbm.at[c, pl.ds(s * _RPT, _RPT)])

        @pl.when(s == _NS - 1)
        def _out_tail():
            pltpu.sync_copy(agg_sh.at[pl.ds(_NS * _RPT, _TAIL)],
                            out_hbm.at[c, pl.ds(_NS * _RPT, _TAIL)])

    return k(x, src, dst, ew, zeros)


def _tc_epilogue(p0, p1, x0, xin, w1):
    blk = 1000

    def body(p0_ref, p1_ref, x0_ref, xin_ref, w1_ref, o_ref):
        agg = p0_ref[...] + p1_ref[...]
        h = (1.0 - _ALPHA) * agg + _ALPHA * x0_ref[...]
        hw = jnp.dot(h, w1_ref[...], preferred_element_type=jnp.float32)
        o_ref[...] = jnp.maximum(
            (1.0 - _BETA) * h + _BETA * hw + xin_ref[...], 0.0)

    return pl.pallas_call(
        body,
        grid=(_N // blk,),
        in_specs=[
            pl.BlockSpec((blk, _D), lambda i: (i, 0)),
            pl.BlockSpec((blk, _D), lambda i: (i, 0)),
            pl.BlockSpec((blk, _D), lambda i: (i, 0)),
            pl.BlockSpec((blk, _D), lambda i: (i, 0)),
            pl.BlockSpec((_D, _D), lambda i: (0, 0)),
        ],
        out_specs=pl.BlockSpec((blk, _D), lambda i: (i, 0)),
        out_shape=jax.ShapeDtypeStruct((_N, _D), jnp.float32),
    )(p0, p1, x0, xin, w1)


def kernel(x, x_0, edge_index, edge_weight, W1):
    src = edge_index[0].astype(jnp.int32)
    dst = edge_index[1].astype(jnp.int32)
    zeros = jnp.zeros((_N, _D), jnp.float32)
    partials = _sc_gather_scatter(x, src, dst, edge_weight, zeros)
    return _tc_epilogue(partials[0], partials[1], x_0, x, W1)


# speedup vs baseline: 8.6902x; 2.1290x over previous
import functools
import math

import jax
import jax.numpy as jnp
from jax import lax
from jax.experimental import pallas as pl
from jax.experimental.pallas import tpu as pltpu
from jax.experimental.pallas import tpu_sc as plsc

_N = 10000
_D = 128
_E = 320000
_ALPHA = 0.1
_BETA = float(math.log(0.5 / 2.0 + 1.0))

_NC = 2
_NS = 16
_NW = _NC * _NS
_L = 16

_EPT = 10240
_S = 128
_NCH = _EPT // _S
_EPT2 = (_NCH + 2) * _S
_E2 = _NW * _EPT

_RPT = 624
_TAIL = _N - _NS * _RPT


def _splat(vec, i):
    idx = jnp.full((_L,), i, jnp.int32)
    dnums = lax.GatherDimensionNumbers(
        offset_dims=(), collapsed_slice_dims=(0,), start_index_map=(0,))
    return lax.gather(vec, idx[:, None], dnums, (1,),
                      mode=lax.GatherScatterMode.PROMISE_IN_BOUNDS)


def _sc_gather_scatter(x, src_flat, dst3, ew_flat, zeros):
    mesh = plsc.VectorSubcoreMesh(
        core_axis_name="c", subcore_axis_name="s",
        num_cores=_NC, num_subcores=_NS)

    @functools.partial(
        pl.kernel,
        out_type=jax.ShapeDtypeStruct((_NC, _N, _D), jnp.float32),
        mesh=mesh,
        scratch_types=[
            pltpu.VMEM((2, _S), jnp.int32),
            pltpu.VMEM((_NCH, _S), jnp.int32),
            pltpu.VMEM((2, _S), jnp.float32),
            pltpu.VMEM((_S, _D), jnp.float32),
            pltpu.VMEM((_S, _D), jnp.float32),
            pltpu.VMEM_SHARED((_N, _D), jnp.float32),
            pltpu.SemaphoreType.DMA,
            pltpu.SemaphoreType.DMA,
            pltpu.SemaphoreType.DMA,
            pltpu.SemaphoreType.DMA,
            pltpu.SemaphoreType.DMA,
        ],
    )
    def k(x_hbm, src_hbm, dst_hbm, ew_hbm, z_hbm, out_hbm,
          src_v, dst_v, w_v, rows0, rows1, agg_sh,
          zsem, isem0, isem1, gsem0, gsem1):
        c = lax.axis_index("c")
        s = lax.axis_index("s")
        gwid = s * _NC + c
        ebase = gwid * _EPT2

        rows = (rows0, rows1)
        isems = (isem0, isem1)
        gsems = (gsem0, gsem1)

        zc = pltpu.async_copy(z_hbm.at[pl.ds(s * _RPT, _RPT)],
                              agg_sh.at[pl.ds(s * _RPT, _RPT)], zsem)

        pltpu.sync_copy(dst_hbm.at[gwid], dst_v)

        def fire_idx(kk, b):
            d0 = pltpu.async_copy(src_hbm.at[pl.ds(ebase + kk * _S, _S)],
                                  src_v.at[b], isems[b])
            d1 = pltpu.async_copy(ew_hbm.at[pl.ds(ebase + kk * _S, _S)],
                                  w_v.at[b], isems[b])
            return d0, d1

        def wait_idx(b):
            pltpu.make_async_copy(src_hbm.at[pl.ds(0, _S)],
                                  src_v.at[b], isems[b]).wait()
            pltpu.make_async_copy(ew_hbm.at[pl.ds(0, _S)],
                                  w_v.at[b], isems[b]).wait()

        def fire_gather(b):
            pltpu.async_copy(x_hbm.at[src_v.at[b]], rows[b], gsems[b])

        def wait_gather(b):
            pltpu.make_async_copy(x_hbm.at[pl.ds(0, _S)],
                                  rows[b], gsems[b]).wait()

        d0, d1 = fire_idx(0, 0)
        d0.wait()
        d1.wait()
        fire_gather(0)
        fire_idx(1, 1)

        zc.wait()

        @pl.when(s == _NS - 1)
        def _zero_tail():
            pltpu.async_copy(z_hbm.at[pl.ds(_NS * _RPT, _TAIL)],
                             agg_sh.at[pl.ds(_NS * _RPT, _TAIL)], zsem).wait()

        plsc.subcore_barrier()

        def scale(b, kk):
            def grp(g, carry):
                w16 = w_v[b, pl.ds(g * _L, _L)]
                for e in range(_L):
                    ws = _splat(w16, e)
                    row = g * _L + e
                    for j in range(_D // _L):
                        rows[b][row, pl.ds(j * _L, _L)] = (
                            rows[b][row, pl.ds(j * _L, _L)] * ws)
                return carry
            lax.fori_loop(0, _S // _L, grp, 0)

        def half(i, b, kk):
            wait_idx(1 - b)
            fire_gather(1 - b)
            wait_gather(b)
            scale(b, kk)
            pltpu.sync_copy(rows[b], agg_sh.at[dst_v.at[kk]], add=True)
            fire_idx(kk + 2, b)

        def body(i, carry):
            half(i, 0, 2 * i)
            half(i, 1, 2 * i + 1)
            return carry

        lax.fori_loop(0, _NCH // 2, body, 0)

        wait_gather(0)
        wait_idx(1)

        plsc.subcore_barrier()
        pltpu.sync_copy(agg_sh.at[pl.ds(s * _RPT, _RPT)],
                        out_hbm.at[c, pl.ds(s * _RPT, _RPT)])

        @pl.when(s == _NS - 1)
        def _out_tail():
            pltpu.sync_copy(agg_sh.at[pl.ds(_NS * _RPT, _TAIL)],
                            out_hbm.at[c, pl.ds(_NS * _RPT, _TAIL)])

    return k(x, src_flat, dst3, ew_flat, zeros)


def _tc_epilogue(p0, p1, x0, xin, w1):
    blk = 1000

    def body(p0_ref, p1_ref, x0_ref, xin_ref, w1_ref, o_ref):
        agg = p0_ref[...] + p1_ref[...]
        h = (1.0 - _ALPHA) * agg + _ALPHA * x0_ref[...]
        hw = jnp.dot(h, w1_ref[...], preferred_element_type=jnp.float32)
        o_ref[...] = jnp.maximum(
            (1.0 - _BETA) * h + _BETA * hw + xin_ref[...], 0.0)

    return pl.pallas_call(
        body,
        grid=(_N // blk,),
        in_specs=[
            pl.BlockSpec((blk, _D), lambda i: (i, 0)),
            pl.BlockSpec((blk, _D), lambda i: (i, 0)),
            pl.BlockSpec((blk, _D), lambda i: (i, 0)),
            pl.BlockSpec((blk, _D), lambda i: (i, 0)),
            pl.BlockSpec((_D, _D), lambda i: (0, 0)),
        ],
        out_specs=pl.BlockSpec((blk, _D), lambda i: (i, 0)),
        out_shape=jax.ShapeDtypeStruct((_N, _D), jnp.float32),
    )(p0, p1, x0, xin, w1)


def kernel(x, x_0, edge_index, edge_weight, W1):
    src = edge_index[0].astype(jnp.int32)
    dst = edge_index[1].astype(jnp.int32)

    pad = _E2 - _E
    fill = (jnp.arange(pad, dtype=jnp.int32) * 13) % _N
    src_p = jnp.concatenate([src, fill]).reshape(_NW, _EPT)
    dst_p = jnp.concatenate([dst, fill]).reshape(_NW, _NCH, _S)
    ew_p = jnp.concatenate(
        [edge_weight, jnp.zeros((pad,), jnp.float32)]).reshape(_NW, _EPT)

    dummy_i = jnp.broadcast_to(
        ((jnp.arange(2 * _S, dtype=jnp.int32) * 37) % _N)[None, :],
        (_NW, 2 * _S))
    dummy_f = jnp.zeros((_NW, 2 * _S), jnp.float32)
    src_flat = jnp.concatenate([src_p, dummy_i], axis=1).reshape(-1)
    ew_flat = jnp.concatenate([ew_p, dummy_f], axis=1).reshape(-1)

    zeros = jnp.zeros((_N, _D), jnp.float32)
    partials = _sc_gather_scatter(x, src_flat, dst_p, ew_flat, zeros)
    return _tc_epilogue(partials[0], partials[1], x_0, x, W1)


# speedup vs baseline: 9.2472x; 1.0641x over previous
import functools
import math

import jax
import jax.numpy as jnp
from jax import lax
from jax.experimental import pallas as pl
from jax.experimental.pallas import tpu as pltpu
from jax.experimental.pallas import tpu_sc as plsc

_N = 10000
_D = 128
_E = 320000
_ALPHA = 0.1
_BETA = float(math.log(0.5 / 2.0 + 1.0))

_NC = 2
_NS = 16
_NW = _NC * _NS
_L = 16

_S = 96
_NCH = 108
_NCHT = _NCH + 2
_EPT = _NCH * _S
_E2 = _NW * _EPT
_NB = 4

_RPT = 624
_TAIL = _N - _NS * _RPT


def _splat(vec, i):
    idx = jnp.full((_L,), i, jnp.int32)
    dnums = lax.GatherDimensionNumbers(
        offset_dims=(), collapsed_slice_dims=(0,), start_index_map=(0,))
    return lax.gather(vec, idx[:, None], dnums, (1,),
                      mode=lax.GatherScatterMode.PROMISE_IN_BOUNDS)


def _sc_gather_scatter(x, pk, wf, zeros):
    mesh = plsc.VectorSubcoreMesh(
        core_axis_name="c", subcore_axis_name="s",
        num_cores=_NC, num_subcores=_NS)

    @functools.partial(
        pl.kernel,
        out_type=jax.ShapeDtypeStruct((_NC, _N, _D), jnp.float32),
        mesh=mesh,
        scratch_types=[
            pltpu.VMEM((_NB, 2, _S), jnp.int32),
            pltpu.VMEM((_NB, _S), jnp.float32),
            pltpu.VMEM((_S, _D), jnp.float32),
            pltpu.VMEM((_S, _D), jnp.float32),
            pltpu.VMEM((_S, _D), jnp.float32),
            pltpu.VMEM((_S, _D), jnp.float32),
            pltpu.VMEM_SHARED((_N, _D), jnp.float32),
            pltpu.SemaphoreType.DMA,
            (pltpu.SemaphoreType.DMA,) * _NB,
            (pltpu.SemaphoreType.DMA,) * _NB,
            (pltpu.SemaphoreType.DMA,) * _NB,
        ],
    )
    def k(x_hbm, pk_hbm, wf_hbm, z_hbm, out_hbm,
          pk_v, w_v, rows0, rows1, rows2, rows3, agg_sh,
          zsem, isems, gsems, ssems):
        c = lax.axis_index("c")
        s = lax.axis_index("s")
        gwid = s * _NC + c
        cbase = gwid * _NCHT

        rows = (rows0, rows1, rows2, rows3)

        zc = pltpu.async_copy(z_hbm.at[pl.ds(s * _RPT, _RPT)],
                              agg_sh.at[pl.ds(s * _RPT, _RPT)], zsem)

        def fire_idx(kk, b):
            pltpu.async_copy(pk_hbm.at[cbase + kk], pk_v.at[b], isems[b])
            pltpu.async_copy(wf_hbm.at[pl.ds((cbase + kk) * _S, _S)],
                             w_v.at[b], isems[b])

        def wait_idx(b):
            pltpu.make_async_copy(pk_hbm.at[0], pk_v.at[b], isems[b]).wait()
            pltpu.make_async_copy(wf_hbm.at[pl.ds(0, _S)], w_v.at[b],
                                  isems[b]).wait()

        def fire_gather(b):
            pltpu.async_copy(x_hbm.at[pk_v.at[b, 0]], rows[b], gsems[b])

        def wait_gather(b):
            pltpu.make_async_copy(x_hbm.at[pl.ds(0, _S)], rows[b],
                                  gsems[b]).wait()

        def fire_scatter(b):
            pltpu.async_copy(rows[b], agg_sh.at[pk_v.at[b, 1]], ssems[b],
                             add=True)

        def wait_scatter(b):
            pltpu.make_async_copy(rows[b], agg_sh.at[pl.ds(0, _S)],
                                  ssems[b]).wait()

        def scale(b, kk):
            def grp(g, carry):
                w16 = w_v[b, pl.ds(g * _L, _L)]
                for e in range(_L):
                    ws = _splat(w16, e)
                    row = g * _L + e
                    for j in range(_D // _L):
                        rows[b][row, pl.ds(j * _L, _L)] = (
                            rows[b][row, pl.ds(j * _L, _L)] * ws)
                return carry
            lax.fori_loop(0, _S // _L, grp, 0)

        fire_idx(0, 0)
        fire_idx(1, 1)
        wait_idx(0)
        fire_gather(0)

        zc.wait()

        @pl.when(s == _NS - 1)
        def _zero_tail():
            pltpu.async_copy(z_hbm.at[pl.ds(_NS * _RPT, _TAIL)],
                             agg_sh.at[pl.ds(_NS * _RPT, _TAIL)], zsem).wait()

        plsc.subcore_barrier()

        def body(i, carry):
            for h in range(_NB):
                kk = _NB * i + h
                b = h
                bn = (h + 1) % _NB
                b2 = (h + 2) % _NB

                if h < 2:
                    @pl.when(kk >= 2)
                    def _ws():
                        wait_scatter(b2)
                else:
                    wait_scatter(b2)
                wait_idx(bn)
                fire_gather(bn)
                wait_gather(b)
                scale(b, kk)
                fire_scatter(b)
                fire_idx(kk + 2, b2)
            return carry

        lax.fori_loop(0, _NCH // _NB, body, 0)

        wait_scatter((_NCH - 2) % _NB)
        wait_scatter((_NCH - 1) % _NB)
        wait_gather(_NCH % _NB)
        wait_idx((_NCH + 1) % _NB)

        plsc.subcore_barrier()
        pltpu.sync_copy(agg_sh.at[pl.ds(s * _RPT, _RPT)],
                        out_hbm.at[c, pl.ds(s * _RPT, _RPT)])

        @pl.when(s == _NS - 1)
        def _out_tail():
            pltpu.sync_copy(agg_sh.at[pl.ds(_NS * _RPT, _TAIL)],
                            out_hbm.at[c, pl.ds(_NS * _RPT, _TAIL)])

    return k(x, pk, wf, zeros)


def _tc_epilogue(p0, p1, x0, xin, w1):
    blk = 1000

    def body(p0_ref, p1_ref, x0_ref, xin_ref, w1_ref, o_ref):
        agg = p0_ref[...] + p1_ref[...]
        h = (1.0 - _ALPHA) * agg + _ALPHA * x0_ref[...]
        hw = jnp.dot(h, w1_ref[...], preferred_element_type=jnp.float32)
        o_ref[...] = jnp.maximum(
            (1.0 - _BETA) * h + _BETA * hw + xin_ref[...], 0.0)

    return pl.pallas_call(
        body,
        grid=(_N // blk,),
        in_specs=[
            pl.BlockSpec((blk, _D), lambda i: (i, 0)),
            pl.BlockSpec((blk, _D), lambda i: (i, 0)),
            pl.BlockSpec((blk, _D), lambda i: (i, 0)),
            pl.BlockSpec((blk, _D), lambda i: (i, 0)),
            pl.BlockSpec((_D, _D), lambda i: (0, 0)),
        ],
        out_specs=pl.BlockSpec((blk, _D), lambda i: (i, 0)),
        out_shape=jax.ShapeDtypeStruct((_N, _D), jnp.float32),
    )(p0, p1, x0, xin, w1)


def kernel(x, x_0, edge_index, edge_weight, W1):
    src = edge_index[0].astype(jnp.int32)
    dst = edge_index[1].astype(jnp.int32)

    pad = _E2 - _E
    fill = (jnp.arange(pad, dtype=jnp.int32) * 13) % _N
    src_p = jnp.concatenate([src, fill]).reshape(_NW, _NCH, _S)
    dst_p = jnp.concatenate([dst, fill]).reshape(_NW, _NCH, _S)
    pk = jnp.stack([src_p, dst_p], axis=2)

    dummy_i = jnp.broadcast_to(
        ((jnp.arange(2 * _S, dtype=jnp.int32) * 37) % _N).reshape(2, _S)[
            None, :, None, :],
        (_NW, 2, 1, _S))
    dummy = jnp.concatenate([dummy_i, dummy_i], axis=2)
    pk = jnp.concatenate([pk, dummy], axis=1).reshape(_NW * _NCHT, 2, _S)

    w_p = jnp.concatenate(
        [edge_weight, jnp.zeros((pad,), jnp.float32)]).reshape(_NW, _NCH, _S)
    wf = jnp.concatenate(
        [w_p, jnp.zeros((_NW, 2, _S), jnp.float32)], axis=1).reshape(-1)

    zeros = jnp.zeros((_N, _D), jnp.float32)
    partials = _sc_gather_scatter(x, pk, wf, zeros)
    return _tc_epilogue(partials[0], partials[1], x_0, x, W1)
